# baseline (speedup 1.0000x reference)
import functools

import jax
import jax.numpy as jnp
from jax import lax
from jax.experimental import pallas as pl
from jax.experimental.pallas import tpu as pltpu
from jax.experimental.pallas import tpu_sc as plsc

N = 10000
E = 320000
D = 128

NC = 2
NS = 16
NW = NC * NS

CHUNK = 128
CPW = 80
E_PAD = NW * CPW * CHUNK
NACC = N + 8

RPS = 632
RPS_LAST = N - RPS * (NS - 1)


def _sc_segment_sum(feat, srcp, dstp):
    mesh = plsc.VectorSubcoreMesh(core_axis_name="c", subcore_axis_name="s")

    @functools.partial(
        pl.kernel,
        out_type=jax.ShapeDtypeStruct((NC * N, D), jnp.float32),
        mesh=mesh,
        scratch_types=[
            pltpu.VMEM((CPW // 2, CHUNK), jnp.int32),
            pltpu.VMEM((CPW // 2, CHUNK), jnp.int32),
            pltpu.VMEM((CHUNK, D), jnp.float32),
            pltpu.VMEM((CHUNK, D), jnp.float32),
            pltpu.VMEM_SHARED((NACC, D), jnp.float32),
            pltpu.SemaphoreType.DMA,
            pltpu.SemaphoreType.DMA,
            pltpu.SemaphoreType.DMA,
            pltpu.SemaphoreType.DMA,
        ],
    )
    def k(feat_hbm, src_hbm, dst_hbm, out_hbm,
          sidx, didx, rows0, rows1, acc, g0, g1, s0, s1):
        c = lax.axis_index("c")
        s = lax.axis_index("s")
        wid = c * NS + s

        @pl.loop(0, CHUNK)
        def _(i):
            @pl.loop(0, D, step=16)
            def _(j):
                rows0.at[i, pl.ds(j, 16)][...] = jnp.zeros((16,), jnp.float32)

        base_r = s * RPS

        def zero_rows(tail):
            @pl.loop(0, 4)
            def _(r):
                pltpu.sync_copy(rows0, acc.at[pl.ds(base_r + r * CHUNK, CHUNK)])
            pltpu.sync_copy(rows0.at[pl.ds(0, tail)],
                            acc.at[pl.ds(base_r + 4 * CHUNK, tail)])

        @pl.when(s < NS - 1)
        def _():
            zero_rows(RPS - 4 * CHUNK)

        @pl.when(s == NS - 1)
        def _():
            zero_rows(RPS_LAST + 8 - 4 * CHUNK)

        plsc.subcore_barrier()

        HALF = CPW // 2
        for h in range(2):
            base = wid * CPW + h * HALF
            pltpu.sync_copy(src_hbm.at[pl.ds(base, HALF)], sidx)
            pltpu.sync_copy(dst_hbm.at[pl.ds(base, HALF)], didx)

            @pl.loop(0, HALF)
            def _(t):
                pltpu.sync_copy(feat_hbm.at[sidx.at[t]], rows0)
                pltpu.sync_copy(rows0, acc.at[didx.at[t]], add=True)

        plsc.subcore_barrier()

        @pl.when(s < NS - 1)
        def _():
            pltpu.sync_copy(acc.at[pl.ds(base_r, RPS)],
                            out_hbm.at[pl.ds(c * N + base_r, RPS)])

        @pl.when(s == NS - 1)
        def _():
            pltpu.sync_copy(acc.at[pl.ds(base_r, RPS_LAST)],
                            out_hbm.at[pl.ds(c * N + base_r, RPS_LAST)])

    return k(feat, srcp, dstp)


def _tc_layer(x, p0, p1, W, b2d, final):
    BR = 1000

    def body(x_ref, p0_ref, p1_ref, w_ref, b_ref, o_ref):
        t = x_ref[...] + p0_ref[...] + p1_ref[...]
        acc = jnp.dot(t, w_ref[...], preferred_element_type=jnp.float32,
                      precision=lax.Precision.HIGHEST) + b_ref[...]
        if final:
            m = jnp.max(acc, axis=1, keepdims=True)
            e = acc - m
            lse = jnp.log(jnp.sum(jnp.exp(e), axis=1, keepdims=True))
            o_ref[...] = e - lse
        else:
            o_ref[...] = jnp.maximum(acc, 0.0)

    return pl.pallas_call(
        body,
        grid=(N // BR,),
        in_specs=[
            pl.BlockSpec((BR, D), lambda i: (i, 0)),
            pl.BlockSpec((BR, D), lambda i: (i, 0)),
            pl.BlockSpec((BR, D), lambda i: (i, 0)),
            pl.BlockSpec((D, D), lambda i: (0, 0)),
            pl.BlockSpec((1, D), lambda i: (0, 0)),
        ],
        out_specs=pl.BlockSpec((BR, D), lambda i: (i, 0)),
        out_shape=jax.ShapeDtypeStruct((N, D), jnp.float32),
    )(x, p0, p1, W, b2d)


def kernel(input_feature, edge_index, W1, b1, W2, b2):
    src = edge_index[0]
    dst = edge_index[1]
    npad = E_PAD - E
    srcp = jnp.concatenate([src, jnp.zeros((npad,), jnp.int32)])
    dstp = jnp.concatenate([dst, jnp.full((npad,), N, jnp.int32)])
    srcp = srcp.reshape(NW * CPW, CHUNK)
    dstp = dstp.reshape(NW * CPW, CHUNK)
    b1_2d = b1.reshape(1, D)
    b2_2d = b2.reshape(1, D)

    p = _sc_segment_sum(input_feature, srcp, dstp)
    h = _tc_layer(input_feature, p[:N], p[N:], W1, b1_2d, final=False)
    q = _sc_segment_sum(h, srcp, dstp)
    return _tc_layer(h, q[:N], q[N:], W2, b2_2d, final=True)


# speedup vs baseline: 2.3736x; 2.3736x over previous
import functools

import jax
import jax.numpy as jnp
from jax import lax
from jax.experimental import pallas as pl
from jax.experimental.pallas import tpu as pltpu
from jax.experimental.pallas import tpu_sc as plsc

N = 10000
E = 320000
D = 128

NC = 2
NS = 16
NW = NC * NS

CHUNK = 128
NCHUNKS = E // CHUNK
CHUNKS_PER_W = NCHUNKS // NW
REM = NCHUNKS - CHUNKS_PER_W * NW

RPS = 632
RPS_LAST = N - RPS * (NS - 1)


def _sc_segment_sum(feat, src, dst):
    mesh = plsc.VectorSubcoreMesh(core_axis_name="c", subcore_axis_name="s")

    @functools.partial(
        pl.kernel,
        out_type=jax.ShapeDtypeStruct((NC * N, D), jnp.float32),
        mesh=mesh,
        scratch_types=[
            pltpu.VMEM((CHUNK,), jnp.int32),
            pltpu.VMEM((CHUNK,), jnp.int32),
            pltpu.VMEM((CHUNK,), jnp.int32),
            pltpu.VMEM((CHUNK,), jnp.int32),
            pltpu.VMEM((CHUNK, D), jnp.float32),
            pltpu.VMEM((CHUNK, D), jnp.float32),
            pltpu.VMEM_SHARED((N, D), jnp.float32),
            pltpu.SemaphoreType.DMA,
            pltpu.SemaphoreType.DMA,
        ],
    )
    def k(feat_hbm, src_hbm, dst_hbm, out_hbm,
          sidx0, sidx1, didx0, didx1, rows0, rows1, acc, g0, g1):
        c = lax.axis_index("c")
        s = lax.axis_index("s")
        wid = c * NS + s

        @pl.loop(0, CHUNK)
        def _(i):
            @pl.loop(0, D, step=16)
            def _(j):
                rows0.at[i, pl.ds(j, 16)][...] = jnp.zeros((16,), jnp.float32)

        base_r = s * RPS

        def zero_rows(tail):
            @pl.loop(0, 4)
            def _(r):
                pltpu.sync_copy(rows0, acc.at[pl.ds(base_r + r * CHUNK, CHUNK)])
            pltpu.sync_copy(rows0.at[pl.ds(0, tail)],
                            acc.at[pl.ds(base_r + 4 * CHUNK, tail)])

        @pl.when(s < NS - 1)
        def _():
            zero_rows(RPS - 4 * CHUNK)

        @pl.when(s == NS - 1)
        def _():
            zero_rows(RPS_LAST - 4 * CHUNK)

        plsc.subcore_barrier()

        def do_pair(cid0):
            e0 = cid0 * CHUNK
            e1 = e0 + CHUNK
            pltpu.sync_copy(src_hbm.at[pl.ds(e0, CHUNK)], sidx0)
            pltpu.sync_copy(src_hbm.at[pl.ds(e1, CHUNK)], sidx1)
            d0 = pltpu.async_copy(feat_hbm.at[sidx0], rows0, g0)
            d1 = pltpu.async_copy(feat_hbm.at[sidx1], rows1, g1)
            pltpu.sync_copy(dst_hbm.at[pl.ds(e0, CHUNK)], didx0)
            pltpu.sync_copy(dst_hbm.at[pl.ds(e1, CHUNK)], didx1)
            d0.wait()
            pltpu.sync_copy(rows0, acc.at[didx0], add=True)
            d1.wait()
            pltpu.sync_copy(rows1, acc.at[didx1], add=True)

        base_c = wid * CHUNKS_PER_W

        @pl.loop(0, CHUNKS_PER_W, step=2)
        def _(t):
            do_pair(base_c + t)

        @pl.when(wid < REM // 2)
        def _():
            do_pair(NW * CHUNKS_PER_W + wid * 2)

        plsc.subcore_barrier()

        @pl.when(s < NS - 1)
        def _():
            pltpu.sync_copy(acc.at[pl.ds(base_r, RPS)],
                            out_hbm.at[pl.ds(c * N + base_r, RPS)])

        @pl.when(s == NS - 1)
        def _():
            pltpu.sync_copy(acc.at[pl.ds(base_r, RPS_LAST)],
                            out_hbm.at[pl.ds(c * N + base_r, RPS_LAST)])

    return k(feat, src, dst)


def _tc_layer(x, p0, p1, W, b2d, final):
    BR = 1000

    def body(x_ref, p0_ref, p1_ref, w_ref, b_ref, o_ref):
        t = x_ref[...] + p0_ref[...] + p1_ref[...]
        acc = jnp.dot(t, w_ref[...], preferred_element_type=jnp.float32,
                      precision=lax.Precision.HIGHEST) + b_ref[...]
        if final:
            m = jnp.max(acc, axis=1, keepdims=True)
            e = acc - m
            lse = jnp.log(jnp.sum(jnp.exp(e), axis=1, keepdims=True))
            o_ref[...] = e - lse
        else:
            o_ref[...] = jnp.maximum(acc, 0.0)

    return pl.pallas_call(
        body,
        grid=(N // BR,),
        in_specs=[
            pl.BlockSpec((BR, D), lambda i: (i, 0)),
            pl.BlockSpec((BR, D), lambda i: (i, 0)),
            pl.BlockSpec((BR, D), lambda i: (i, 0)),
            pl.BlockSpec((D, D), lambda i: (0, 0)),
            pl.BlockSpec((1, D), lambda i: (0, 0)),
        ],
        out_specs=pl.BlockSpec((BR, D), lambda i: (i, 0)),
        out_shape=jax.ShapeDtypeStruct((N, D), jnp.float32),
    )(x, p0, p1, W, b2d)


def kernel(input_feature, edge_index, W1, b1, W2, b2):
    src = edge_index[0]
    dst = edge_index[1]
    b1_2d = b1.reshape(1, D)
    b2_2d = b2.reshape(1, D)

    p = _sc_segment_sum(input_feature, src, dst)
    h = _tc_layer(input_feature, p[:N], p[N:], W1, b1_2d, final=False)
    q = _sc_segment_sum(h, src, dst)
    return _tc_layer(h, q[:N], q[N:], W2, b2_2d, final=True)


# speedup vs baseline: 2.8411x; 1.1970x over previous
import functools

import jax
import jax.numpy as jnp
from jax import lax
from jax.experimental import pallas as pl
from jax.experimental.pallas import tpu as pltpu
from jax.experimental.pallas import tpu_sc as plsc

N = 10000
E = 320000
D = 128

NC = 2
NS = 16
NW = NC * NS

CHUNK = 128
NCHUNKS = E // CHUNK
CHUNKS_PER_W = NCHUNKS // NW
REM = NCHUNKS - CHUNKS_PER_W * NW

RPS = 632
RPS_LAST = N - RPS * (NS - 1)


def _sc_segment_sum(feat, src, dst):
    mesh = plsc.VectorSubcoreMesh(core_axis_name="c", subcore_axis_name="s")

    @functools.partial(
        pl.kernel,
        out_type=jax.ShapeDtypeStruct((NC * N, D), jnp.float32),
        mesh=mesh,
        scratch_types=[
            pltpu.VMEM((CHUNK,), jnp.int32),
            pltpu.VMEM((CHUNK,), jnp.int32),
            pltpu.VMEM((CHUNK,), jnp.int32),
            pltpu.VMEM((CHUNK,), jnp.int32),
            pltpu.VMEM((CHUNK, D), jnp.float32),
            pltpu.VMEM((CHUNK, D), jnp.float32),
            pltpu.VMEM_SHARED((N, D), jnp.float32),
            pltpu.SemaphoreType.DMA,
            pltpu.SemaphoreType.DMA,
        ],
    )
    def k(feat_hbm, src_hbm, dst_hbm, out_hbm,
          sidx0, sidx1, didx0, didx1, rows0, rows1, acc, g0, g1):
        c = lax.axis_index("c")
        s = lax.axis_index("s")
        wid = c * NS + s

        @pl.loop(0, CHUNK)
        def _(i):
            @pl.loop(0, D, step=16)
            def _(j):
                rows0.at[i, pl.ds(j, 16)][...] = jnp.zeros((16,), jnp.float32)

        base_r = s * RPS

        def zero_rows(tail):
            @pl.loop(0, 4)
            def _(r):
                pltpu.sync_copy(rows0, acc.at[pl.ds(base_r + r * CHUNK, CHUNK)])
            pltpu.sync_copy(rows0.at[pl.ds(0, tail)],
                            acc.at[pl.ds(base_r + 4 * CHUNK, tail)])

        @pl.when(s < NS - 1)
        def _():
            zero_rows(RPS - 4 * CHUNK)

        @pl.when(s == NS - 1)
        def _():
            zero_rows(RPS_LAST - 4 * CHUNK)

        plsc.subcore_barrier()

        base_c = wid * CHUNKS_PER_W
        bufs = ((sidx0, didx0, rows0, g0), (sidx1, didx1, rows1, g1))

        def load_idx(cid, sb, db):
            e0 = cid * CHUNK
            pltpu.sync_copy(src_hbm.at[pl.ds(e0, CHUNK)], sb)
            pltpu.sync_copy(dst_hbm.at[pl.ds(e0, CHUNK)], db)

        load_idx(base_c, sidx0, didx0)
        pltpu.async_copy(feat_hbm.at[sidx0], rows0, g0)

        @pl.loop(0, CHUNKS_PER_W - 2, step=2)
        def _(t):
            for st in range(2):
                sb, db, rb, gs = bufs[st]
                nsb, ndb, nrb, ngs = bufs[1 - st]
                load_idx(base_c + t + st + 1, nsb, ndb)
                pltpu.async_copy(feat_hbm.at[nsb], nrb, ngs)
                pltpu.make_async_copy(feat_hbm.at[sb], rb, gs).wait()
                pltpu.sync_copy(rb, acc.at[db], add=True)

        load_idx(base_c + CHUNKS_PER_W - 1, sidx1, didx1)
        pltpu.async_copy(feat_hbm.at[sidx1], rows1, g1)
        pltpu.make_async_copy(feat_hbm.at[sidx0], rows0, g0).wait()
        pltpu.sync_copy(rows0, acc.at[didx0], add=True)
        pltpu.make_async_copy(feat_hbm.at[sidx1], rows1, g1).wait()
        pltpu.sync_copy(rows1, acc.at[didx1], add=True)

        @pl.when(wid < REM // 2)
        def _():
            c0 = NW * CHUNKS_PER_W + wid * 2
            load_idx(c0, sidx0, didx0)
            d0 = pltpu.async_copy(feat_hbm.at[sidx0], rows0, g0)
            load_idx(c0 + 1, sidx1, didx1)
            d1 = pltpu.async_copy(feat_hbm.at[sidx1], rows1, g1)
            d0.wait()
            pltpu.sync_copy(rows0, acc.at[didx0], add=True)
            d1.wait()
            pltpu.sync_copy(rows1, acc.at[didx1], add=True)

        plsc.subcore_barrier()

        @pl.when(s < NS - 1)
        def _():
            pltpu.sync_copy(acc.at[pl.ds(base_r, RPS)],
                            out_hbm.at[pl.ds(c * N + base_r, RPS)])

        @pl.when(s == NS - 1)
        def _():
            pltpu.sync_copy(acc.at[pl.ds(base_r, RPS_LAST)],
                            out_hbm.at[pl.ds(c * N + base_r, RPS_LAST)])

    return k(feat, src, dst)


def _tc_layer(x, p0, p1, W, b2d, final):
    BR = 1000

    def body(x_ref, p0_ref, p1_ref, w_ref, b_ref, o_ref):
        t = x_ref[...] + p0_ref[...] + p1_ref[...]
        acc = jnp.dot(t, w_ref[...], preferred_element_type=jnp.float32,
                      precision=lax.Precision.HIGHEST) + b_ref[...]
        if final:
            m = jnp.max(acc, axis=1, keepdims=True)
            e = acc - m
            lse = jnp.log(jnp.sum(jnp.exp(e), axis=1, keepdims=True))
            o_ref[...] = e - lse
        else:
            o_ref[...] = jnp.maximum(acc, 0.0)

    return pl.pallas_call(
        body,
        grid=(N // BR,),
        in_specs=[
            pl.BlockSpec((BR, D), lambda i: (i, 0)),
            pl.BlockSpec((BR, D), lambda i: (i, 0)),
            pl.BlockSpec((BR, D), lambda i: (i, 0)),
            pl.BlockSpec((D, D), lambda i: (0, 0)),
            pl.BlockSpec((1, D), lambda i: (0, 0)),
        ],
        out_specs=pl.BlockSpec((BR, D), lambda i: (i, 0)),
        out_shape=jax.ShapeDtypeStruct((N, D), jnp.float32),
    )(x, p0, p1, W, b2d)


def kernel(input_feature, edge_index, W1, b1, W2, b2):
    src = edge_index[0]
    dst = edge_index[1]
    b1_2d = b1.reshape(1, D)
    b2_2d = b2.reshape(1, D)

    p = _sc_segment_sum(input_feature, src, dst)
    h = _tc_layer(input_feature, p[:N], p[N:], W1, b1_2d, final=False)
    q = _sc_segment_sum(h, src, dst)
    return _tc_layer(h, q[:N], q[N:], W2, b2_2d, final=True)


# speedup vs baseline: 2.8446x; 1.0012x over previous
import functools

import jax
import jax.numpy as jnp
from jax import lax
from jax.experimental import pallas as pl
from jax.experimental.pallas import tpu as pltpu
from jax.experimental.pallas import tpu_sc as plsc

N = 10000
E = 320000
D = 128

NC = 2
NS = 16
NW = NC * NS

CHUNK = 128
NCHUNKS = E // CHUNK
CHUNKS_PER_W = NCHUNKS // NW
REM = NCHUNKS - CHUNKS_PER_W * NW

RPS = 632
RPS_LAST = N - RPS * (NS - 1)


def _sc_segment_sum(feat, src, dst):
    mesh = plsc.VectorSubcoreMesh(core_axis_name="c", subcore_axis_name="s")

    @functools.partial(
        pl.kernel,
        out_type=jax.ShapeDtypeStruct((NC * N, D), jnp.float32),
        mesh=mesh,
        scratch_types=[
            pltpu.VMEM((CHUNK,), jnp.int32),
            pltpu.VMEM((CHUNK,), jnp.int32),
            pltpu.VMEM((CHUNK,), jnp.int32),
            pltpu.VMEM((CHUNK,), jnp.int32),
            pltpu.VMEM((CHUNK, D), jnp.float32),
            pltpu.VMEM((CHUNK, D), jnp.float32),
            pltpu.VMEM_SHARED((N, D), jnp.float32),
            pltpu.SemaphoreType.DMA,
            pltpu.SemaphoreType.DMA,
            pltpu.SemaphoreType.DMA,
            pltpu.SemaphoreType.DMA,
        ],
    )
    def k(feat_hbm, src_hbm, dst_hbm, out_hbm,
          sidx0, sidx1, didx0, didx1, rows0, rows1, acc, g0, g1, s0, s1):
        c = lax.axis_index("c")
        s = lax.axis_index("s")
        wid = c * NS + s

        @pl.loop(0, CHUNK)
        def _(i):
            @pl.loop(0, D, step=16)
            def _(j):
                rows0.at[i, pl.ds(j, 16)][...] = jnp.zeros((16,), jnp.float32)

        base_r = s * RPS

        def zero_rows(tail):
            @pl.loop(0, 4)
            def _(r):
                pltpu.sync_copy(rows0, acc.at[pl.ds(base_r + r * CHUNK, CHUNK)])
            pltpu.sync_copy(rows0.at[pl.ds(0, tail)],
                            acc.at[pl.ds(base_r + 4 * CHUNK, tail)])

        @pl.when(s < NS - 1)
        def _():
            zero_rows(RPS - 4 * CHUNK)

        @pl.when(s == NS - 1)
        def _():
            zero_rows(RPS_LAST - 4 * CHUNK)

        plsc.subcore_barrier()

        base_c = wid * CHUNKS_PER_W
        bufs = ((sidx0, didx0, rows0, g0, s0), (sidx1, didx1, rows1, g1, s1))

        def load_idx(cid, sb, db):
            e0 = cid * CHUNK
            pltpu.sync_copy(src_hbm.at[pl.ds(e0, CHUNK)], sb)
            pltpu.sync_copy(dst_hbm.at[pl.ds(e0, CHUNK)], db)

        load_idx(base_c, sidx0, didx0)
        pltpu.async_copy(feat_hbm.at[sidx0], rows0, g0)
        load_idx(base_c + 1, sidx1, didx1)
        pltpu.async_copy(feat_hbm.at[sidx1], rows1, g1)
        pltpu.make_async_copy(feat_hbm.at[sidx0], rows0, g0).wait()
        pltpu.async_copy(rows0, acc.at[didx0], s0, add=True)

        @pl.loop(1, CHUNKS_PER_W - 1, step=2)
        def _(t):
            for st in range(2):
                sb, db, rb, gs, ss = bufs[(1 + st) % 2]
                osb, odb, orb, ogs, oss = bufs[st % 2]
                pltpu.make_async_copy(orb, acc.at[odb], oss).wait()
                load_idx(base_c + t + st + 1, osb, odb)
                pltpu.async_copy(feat_hbm.at[osb], orb, ogs)
                pltpu.make_async_copy(feat_hbm.at[sb], rb, gs).wait()
                pltpu.async_copy(rb, acc.at[db], ss, add=True)

        pltpu.make_async_copy(rows0, acc.at[didx0], s0).wait()
        pltpu.make_async_copy(feat_hbm.at[sidx1], rows1, g1).wait()
        pltpu.sync_copy(rows1, acc.at[didx1], add=True)

        @pl.when(wid < REM // 2)
        def _():
            c0 = NW * CHUNKS_PER_W + wid * 2
            load_idx(c0, sidx0, didx0)
            d0 = pltpu.async_copy(feat_hbm.at[sidx0], rows0, g0)
            load_idx(c0 + 1, sidx1, didx1)
            d1 = pltpu.async_copy(feat_hbm.at[sidx1], rows1, g1)
            d0.wait()
            pltpu.sync_copy(rows0, acc.at[didx0], add=True)
            d1.wait()
            pltpu.sync_copy(rows1, acc.at[didx1], add=True)

        plsc.subcore_barrier()

        @pl.when(s < NS - 1)
        def _():
            pltpu.sync_copy(acc.at[pl.ds(base_r, RPS)],
                            out_hbm.at[pl.ds(c * N + base_r, RPS)])

        @pl.when(s == NS - 1)
        def _():
            pltpu.sync_copy(acc.at[pl.ds(base_r, RPS_LAST)],
                            out_hbm.at[pl.ds(c * N + base_r, RPS_LAST)])

    return k(feat, src, dst)


def _tc_layer(x, p0, p1, W, b2d, final):
    BR = 1000

    def body(x_ref, p0_ref, p1_ref, w_ref, b_ref, o_ref):
        t = x_ref[...] + p0_ref[...] + p1_ref[...]
        acc = jnp.dot(t, w_ref[...], preferred_element_type=jnp.float32,
                      precision=lax.Precision.HIGHEST) + b_ref[...]
        if final:
            m = jnp.max(acc, axis=1, keepdims=True)
            e = acc - m
            lse = jnp.log(jnp.sum(jnp.exp(e), axis=1, keepdims=True))
            o_ref[...] = e - lse
        else:
            o_ref[...] = jnp.maximum(acc, 0.0)

    return pl.pallas_call(
        body,
        grid=(N // BR,),
        in_specs=[
            pl.BlockSpec((BR, D), lambda i: (i, 0)),
            pl.BlockSpec((BR, D), lambda i: (i, 0)),
            pl.BlockSpec((BR, D), lambda i: (i, 0)),
            pl.BlockSpec((D, D), lambda i: (0, 0)),
            pl.BlockSpec((1, D), lambda i: (0, 0)),
        ],
        out_specs=pl.BlockSpec((BR, D), lambda i: (i, 0)),
        out_shape=jax.ShapeDtypeStruct((N, D), jnp.float32),
    )(x, p0, p1, W, b2d)


def kernel(input_feature, edge_index, W1, b1, W2, b2):
    src = edge_index[0]
    dst = edge_index[1]
    b1_2d = b1.reshape(1, D)
    b2_2d = b2.reshape(1, D)

    p = _sc_segment_sum(input_feature, src, dst)
    h = _tc_layer(input_feature, p[:N], p[N:], W1, b1_2d, final=False)
    q = _sc_segment_sum(h, src, dst)
    return _tc_layer(h, q[:N], q[N:], W2, b2_2d, final=True)


# speedup vs baseline: 3.6726x; 1.2911x over previous
import functools

import jax
import jax.numpy as jnp
from jax import lax
from jax.experimental import pallas as pl
from jax.experimental.pallas import tpu as pltpu
from jax.experimental.pallas import tpu_sc as plsc

N = 10000
E = 320000
D = 128

NC = 2
NS = 16
NW = NC * NS

CHUNK = 128
NCHUNKS = E // CHUNK
CHUNKS_PER_W = NCHUNKS // NW
REM = NCHUNKS - CHUNKS_PER_W * NW

RPS = 632
RPS_LAST = N - RPS * (NS - 1)


def _sc_segment_sum(feat, src, dst):
    mesh = plsc.VectorSubcoreMesh(core_axis_name="c", subcore_axis_name="s")

    @functools.partial(
        pl.kernel,
        out_type=jax.ShapeDtypeStruct((NC * N, D), jnp.float32),
        mesh=mesh,
        scratch_types=(
            [pltpu.VMEM((CHUNK,), jnp.int32)] * 8 +
            [
                pltpu.VMEM((CHUNK, D), jnp.float32),
                pltpu.VMEM((CHUNK, D), jnp.float32),
                pltpu.VMEM_SHARED((N, D), jnp.float32),
            ] +
            [pltpu.SemaphoreType.DMA] * 8
        ),
    )
    def k(feat_hbm, src_hbm, dst_hbm, out_hbm,
          sidx0, sidx1, sidx2, sidx3, didx0, didx1, didx2, didx3,
          rows0, rows1, acc, i0, i1, i2, i3, g0, g1, s0, s1):
        c = lax.axis_index("c")
        s = lax.axis_index("s")
        wid = c * NS + s

        @pl.loop(0, CHUNK)
        def _(i):
            @pl.loop(0, D, step=16)
            def _(j):
                rows0.at[i, pl.ds(j, 16)][...] = jnp.zeros((16,), jnp.float32)

        base_r = s * RPS

        def zero_rows(tail):
            @pl.loop(0, 4)
            def _(r):
                pltpu.sync_copy(rows0, acc.at[pl.ds(base_r + r * CHUNK, CHUNK)])
            pltpu.sync_copy(rows0.at[pl.ds(0, tail)],
                            acc.at[pl.ds(base_r + 4 * CHUNK, tail)])

        @pl.when(s < NS - 1)
        def _():
            zero_rows(RPS - 4 * CHUNK)

        @pl.when(s == NS - 1)
        def _():
            zero_rows(RPS_LAST - 4 * CHUNK)

        plsc.subcore_barrier()

        base_c = wid * CHUNKS_PER_W
        SIDX = (sidx0, sidx1, sidx2, sidx3)
        DIDX = (didx0, didx1, didx2, didx3)
        ISEM = (i0, i1, i2, i3)
        ROWS = (rows0, rows1)
        GSEM = (g0, g1)
        SSEM = (s0, s1)

        def idx_start(u, q):
            e0 = (base_c + u) * CHUNK
            pltpu.async_copy(src_hbm.at[pl.ds(e0, CHUNK)], SIDX[q], ISEM[q])
            pltpu.async_copy(dst_hbm.at[pl.ds(e0, CHUNK)], DIDX[q], ISEM[q])

        def idx_wait(u, q):
            e0 = (base_c + u) * CHUNK
            pltpu.make_async_copy(src_hbm.at[pl.ds(e0, CHUNK)], SIDX[q],
                                  ISEM[q]).wait()
            pltpu.make_async_copy(dst_hbm.at[pl.ds(e0, CHUNK)], DIDX[q],
                                  ISEM[q]).wait()

        def gather_start(q, r):
            pltpu.async_copy(feat_hbm.at[SIDX[q]], ROWS[r], GSEM[r])

        def gather_wait(q, r):
            pltpu.make_async_copy(feat_hbm.at[SIDX[q]], ROWS[r],
                                  GSEM[r]).wait()

        def scatter_start(q, r):
            pltpu.async_copy(ROWS[r], acc.at[DIDX[q]], SSEM[r], add=True)

        def scatter_wait(q, r):
            pltpu.make_async_copy(ROWS[r], acc.at[DIDX[q]], SSEM[r]).wait()

        idx_start(0, 0)
        idx_start(1, 1)
        idx_wait(0, 0)
        gather_start(0, 0)
        idx_wait(1, 1)
        gather_start(1, 1)
        idx_start(2, 2)
        gather_wait(0, 0)
        scatter_start(0, 0)
        scatter_wait(0, 0)
        idx_wait(2, 2)
        gather_start(2, 0)
        idx_start(3, 3)
        gather_wait(1, 1)
        scatter_start(1, 1)

        @pl.loop(2, CHUNKS_PER_W, step=4)
        def _(t):
            for st in range(4):
                q = (2 + st) % 4
                r = (2 + st) % 2
                qn = (q + 1) % 4
                qp = (q + 2) % 4
                qm = (q + 3) % 4
                rn = (r + 1) % 2
                u = t + st
                scatter_wait(qm, rn)
                idx_wait(u + 1, qn)
                gather_start(qn, rn)
                idx_start(u + 2, qp)
                gather_wait(q, r)
                scatter_start(q, r)

        scatter_wait(1, 1)
        gather_wait(2, 0)
        idx_wait(CHUNKS_PER_W + 1, 3)

        @pl.when(wid < REM // 2)
        def _():
            e0 = (NW * CHUNKS_PER_W + wid * 2) * CHUNK
            pltpu.sync_copy(src_hbm.at[pl.ds(e0, CHUNK)], sidx0)
            pltpu.sync_copy(dst_hbm.at[pl.ds(e0, CHUNK)], didx0)
            d0 = pltpu.async_copy(feat_hbm.at[sidx0], rows0, g0)
            pltpu.sync_copy(src_hbm.at[pl.ds(e0 + CHUNK, CHUNK)], sidx1)
            pltpu.sync_copy(dst_hbm.at[pl.ds(e0 + CHUNK, CHUNK)], didx1)
            d1 = pltpu.async_copy(feat_hbm.at[sidx1], rows1, g1)
            d0.wait()
            pltpu.sync_copy(rows0, acc.at[didx0], add=True)
            d1.wait()
            pltpu.sync_copy(rows1, acc.at[didx1], add=True)

        plsc.subcore_barrier()

        @pl.when(s < NS - 1)
        def _():
            pltpu.sync_copy(acc.at[pl.ds(base_r, RPS)],
                            out_hbm.at[pl.ds(c * N + base_r, RPS)])

        @pl.when(s == NS - 1)
        def _():
            pltpu.sync_copy(acc.at[pl.ds(base_r, RPS_LAST)],
                            out_hbm.at[pl.ds(c * N + base_r, RPS_LAST)])

    return k(feat, src, dst)


def _tc_layer(x, p0, p1, W, b2d, final):
    BR = 1000

    def body(x_ref, p0_ref, p1_ref, w_ref, b_ref, o_ref):
        t = x_ref[...] + p0_ref[...] + p1_ref[...]
        acc = jnp.dot(t, w_ref[...], preferred_element_type=jnp.float32,
                      precision=lax.Precision.HIGHEST) + b_ref[...]
        if final:
            m = jnp.max(acc, axis=1, keepdims=True)
            e = acc - m
            lse = jnp.log(jnp.sum(jnp.exp(e), axis=1, keepdims=True))
            o_ref[...] = e - lse
        else:
            o_ref[...] = jnp.maximum(acc, 0.0)

    return pl.pallas_call(
        body,
        grid=(N // BR,),
        in_specs=[
            pl.BlockSpec((BR, D), lambda i: (i, 0)),
            pl.BlockSpec((BR, D), lambda i: (i, 0)),
            pl.BlockSpec((BR, D), lambda i: (i, 0)),
            pl.BlockSpec((D, D), lambda i: (0, 0)),
            pl.BlockSpec((1, D), lambda i: (0, 0)),
        ],
        out_specs=pl.BlockSpec((BR, D), lambda i: (i, 0)),
        out_shape=jax.ShapeDtypeStruct((N, D), jnp.float32),
    )(x, p0, p1, W, b2d)


def kernel(input_feature, edge_index, W1, b1, W2, b2):
    src = edge_index[0]
    dst = edge_index[1]
    b1_2d = b1.reshape(1, D)
    b2_2d = b2.reshape(1, D)

    p = _sc_segment_sum(input_feature, src, dst)
    h = _tc_layer(input_feature, p[:N], p[N:], W1, b1_2d, final=False)
    q = _sc_segment_sum(h, src, dst)
    return _tc_layer(h, q[:N], q[N:], W2, b2_2d, final=True)
